# Initial kernel scaffold; baseline (speedup 1.0000x reference)
import jax
import jax.numpy as jnp
from jax.experimental import pallas as pl


def kernel(x, edge_index, edge_attr, W_nl1, b_nl1, W_el, b_el, W_nm1, b_nm1, W_nm2, b_nm2, W_nl2, b_nl2, W_msg, b_msg, W_em, b_em):
    raise NotImplementedError("write your pallas kernel here")


# speedup vs baseline: 3.7072x; 3.7072x over previous
import jax
import jax.numpy as jnp
from jax import lax
from jax.experimental import pallas as pl
from jax.experimental.pallas import tpu as pltpu
from jax.experimental.pallas import tpu_sc as plsc

N_NODES = 10000
N_EDGES = 320000
F = 128
H = 64

NC = 2
NS = 16
NW = NC * NS
PER_TILE = N_EDGES // NW

G_CHUNK = 400
S_CHUNK = 200


def _zt_body(x_ref, w_ref, b_ref, o_ref):
    o_ref[...] = (
        jnp.dot(x_ref[...], w_ref[...], preferred_element_type=jnp.float32)
        + b_ref[...]
    )


def _edge_body(g_ref, ea_ref, b_ref, o_ref):
    o_ref[...] = g_ref[...] + jnp.dot(
        ea_ref[...], b_ref[...], preferred_element_type=jnp.float32
    )


def _node_body(x_ref, s_ref, c_ref, d_ref, c3_ref, o_ref):
    msg = s_ref[0] + s_ref[1]
    o_ref[...] = (
        jnp.dot(x_ref[...], c_ref[...], preferred_element_type=jnp.float32)
        + jnp.dot(msg, d_ref[...], preferred_element_type=jnp.float32)
        + c3_ref[...]
    )


def _gather_body(table_hbm, idx_hbm, out_hbm, idx_v, rows_v, sem):
    c = lax.axis_index("c")
    s = lax.axis_index("s")
    wid = s * NC + c
    base = wid * PER_TILE

    def step(k, carry):
        off = base + k * G_CHUNK
        pltpu.sync_copy(idx_hbm.at[pl.ds(off, G_CHUNK)], idx_v)
        pltpu.async_copy(table_hbm.at[idx_v], rows_v, sem).wait()
        pltpu.sync_copy(rows_v, out_hbm.at[pl.ds(off, G_CHUNK)])
        return carry

    lax.fori_loop(0, PER_TILE // G_CHUNK, step, 0)


def _scatter_body(e_hbm, dst_hbm, zero_hbm, out_hbm, idx_v, rows_v, acc, sem):
    c = lax.axis_index("c")
    s = lax.axis_index("s")
    wid = s * NC + c
    base = wid * PER_TILE

    @pl.when(s == 0)
    def _():
        pltpu.sync_copy(zero_hbm, acc)

    plsc.subcore_barrier()

    def step(k, carry):
        off = base + k * S_CHUNK
        pltpu.sync_copy(dst_hbm.at[pl.ds(off, S_CHUNK)], idx_v)
        pltpu.sync_copy(e_hbm.at[pl.ds(off, S_CHUNK)], rows_v)
        pltpu.sync_copy(rows_v, acc.at[idx_v], add=True)
        return carry

    lax.fori_loop(0, PER_TILE // S_CHUNK, step, 0)
    plsc.subcore_barrier()

    @pl.when(s == 0)
    def _():
        pltpu.sync_copy(acc, out_hbm.at[c])


def kernel(x, edge_index, edge_attr, W_nl1, b_nl1, W_el, b_el, W_nm1, b_nm1,
           W_nm2, b_nm2, W_nl2, b_nl2, W_msg, b_msg, W_em, b_em):
    src = edge_index[0]
    dst = edge_index[1]

    M = W_nm1.T @ W_nm2.T
    c2 = b_nm1 @ W_nm2.T + b_nm2
    A1 = W_nl1.T @ M
    Bw = W_el.T @ M
    bias_z = (b_nl1 + b_el) @ M + c2
    Cw = W_nl2.T @ W_em.T
    Dw = W_msg.T @ W_em.T
    c3 = (b_nl2 + b_msg) @ W_em.T + b_em

    zt = pl.pallas_call(
        _zt_body,
        out_shape=jax.ShapeDtypeStruct((N_NODES, F), jnp.float32),
    )(x, A1, bias_z[None, :])

    mesh = plsc.VectorSubcoreMesh(core_axis_name="c", subcore_axis_name="s")

    gathered = pl.kernel(
        _gather_body,
        out_type=jax.ShapeDtypeStruct((N_EDGES, F), jnp.float32),
        mesh=mesh,
        scratch_types=[
            pltpu.VMEM((G_CHUNK,), jnp.int32),
            pltpu.VMEM((G_CHUNK, F), jnp.float32),
            pltpu.SemaphoreType.DMA,
        ],
    )(zt, src)

    EB = 2000
    e_new = pl.pallas_call(
        _edge_body,
        grid=(N_EDGES // EB,),
        in_specs=[
            pl.BlockSpec((EB, F), lambda i: (i, 0)),
            pl.BlockSpec((EB, F), lambda i: (i, 0)),
            pl.BlockSpec((F, F), lambda i: (0, 0)),
        ],
        out_specs=pl.BlockSpec((EB, F), lambda i: (i, 0)),
        out_shape=jax.ShapeDtypeStruct((N_EDGES, F), jnp.float32),
    )(gathered, edge_attr, Bw)

    zeros = jnp.zeros((N_NODES, F), jnp.float32)
    parts = pl.kernel(
        _scatter_body,
        out_type=jax.ShapeDtypeStruct((NC, N_NODES, F), jnp.float32),
        mesh=mesh,
        scratch_types=[
            pltpu.VMEM((S_CHUNK,), jnp.int32),
            pltpu.VMEM((S_CHUNK, F), jnp.float32),
            pltpu.VMEM_SHARED((N_NODES, F), jnp.float32),
            pltpu.SemaphoreType.DMA,
        ],
    )(e_new, dst, zeros)

    NB = 2000
    x_new = pl.pallas_call(
        _node_body,
        grid=(N_NODES // NB,),
        in_specs=[
            pl.BlockSpec((NB, F), lambda i: (i, 0)),
            pl.BlockSpec((NC, NB, F), lambda i: (0, i, 0)),
            pl.BlockSpec((F, F), lambda i: (0, 0)),
            pl.BlockSpec((F, F), lambda i: (0, 0)),
            pl.BlockSpec((1, F), lambda i: (0, 0)),
        ],
        out_specs=pl.BlockSpec((NB, F), lambda i: (i, 0)),
        out_shape=jax.ShapeDtypeStruct((N_NODES, F), jnp.float32),
    )(x, parts, Cw, Dw, c3[None, :])

    return (e_new, x_new)
